# baseline (speedup 1.0000x reference)
import functools

import jax
import jax.numpy as jnp
from jax import lax
from jax.experimental import pallas as pl
from jax.experimental.pallas import tpu as pltpu
from jax.experimental.pallas import tpu_sc as plsc

N = 10000
E = 160000
E_TOT = E + N
NW = 32
NB = 42
TPW = NB * 128
E_PAD = NW * TPW
N_PAD = 10240
CW = 64
R_ROWS = 2 * N + 160
RPT = R_ROWS // 16
MM_ROWS = 200

_SC_MESH = plsc.VectorSubcoreMesh(core_axis_name="c", subcore_axis_name="s")
_SC_PARAMS = pltpu.CompilerParams(needs_layout_passes=False,
                                  use_tc_tiling_on_sc=False)


def _mm_body(combine, an_ref, ap_ref, adp_ref, den_ref, b_ref, w_ref,
             as_ref, ad_ref, u_ref, aso_ref, ado_ref):
    if combine:
        adp = adp_ref[...]
        z = (jnp.exp(0.2 * adp) * an_ref[...] +
             jnp.exp(adp) * ap_ref[...]) / den_ref[...]
        z = jnp.maximum(z + b_ref[...], 0.0)
    else:
        z = an_ref[...]
    h = jnp.dot(z, w_ref[...], preferred_element_type=jnp.float32,
                precision=lax.Precision.HIGHEST)
    aso = jnp.dot(h, as_ref[...], preferred_element_type=jnp.float32,
                  precision=lax.Precision.HIGHEST)
    ado = jnp.dot(h, ad_ref[...], preferred_element_type=jnp.float32,
                  precision=lax.Precision.HIGHEST)
    u_ref[0] = h * jnp.exp(0.2 * aso)
    u_ref[1] = h * jnp.exp(aso)
    aso_ref[...] = aso
    ado_ref[...] = ado


def _matmul_first(z, W, a_s, a_d):
    K, H = W.shape
    return pl.pallas_call(
        functools.partial(_mm_body, False),
        grid=(N // MM_ROWS,),
        in_specs=[
            pl.BlockSpec((MM_ROWS, K), lambda i: (i, 0)),
            pl.BlockSpec((MM_ROWS, K), lambda i: (i, 0)),
            pl.BlockSpec((MM_ROWS, 1), lambda i: (i, 0)),
            pl.BlockSpec((MM_ROWS, 1), lambda i: (i, 0)),
            pl.BlockSpec((1, K), lambda i: (0, 0)),
            pl.BlockSpec((K, H), lambda i: (0, 0)),
            pl.BlockSpec((H, 1), lambda i: (0, 0)),
            pl.BlockSpec((H, 1), lambda i: (0, 0)),
        ],
        out_specs=[
            pl.BlockSpec((2, MM_ROWS, H), lambda i: (0, i, 0)),
            pl.BlockSpec((MM_ROWS, 1), lambda i: (i, 0)),
            pl.BlockSpec((MM_ROWS, 1), lambda i: (i, 0)),
        ],
        out_shape=[
            jax.ShapeDtypeStruct((2, N, H), jnp.float32),
            jax.ShapeDtypeStruct((N, 1), jnp.float32),
            jax.ShapeDtypeStruct((N, 1), jnp.float32),
        ],
    )(z, z, jnp.zeros((N, 1), jnp.float32), jnp.ones((N, 1), jnp.float32),
      jnp.zeros((1, K), jnp.float32), W, a_s, a_d)


def _matmul_mid(an, ap, adp, den, b, W, a_s, a_d):
    K, H = W.shape
    return pl.pallas_call(
        functools.partial(_mm_body, True),
        grid=(N // MM_ROWS,),
        in_specs=[
            pl.BlockSpec((MM_ROWS, K), lambda i: (i, 0)),
            pl.BlockSpec((MM_ROWS, K), lambda i: (i, 0)),
            pl.BlockSpec((MM_ROWS, 1), lambda i: (i, 0)),
            pl.BlockSpec((MM_ROWS, 1), lambda i: (i, 0)),
            pl.BlockSpec((1, K), lambda i: (0, 0)),
            pl.BlockSpec((K, H), lambda i: (0, 0)),
            pl.BlockSpec((H, 1), lambda i: (0, 0)),
            pl.BlockSpec((H, 1), lambda i: (0, 0)),
        ],
        out_specs=[
            pl.BlockSpec((2, MM_ROWS, H), lambda i: (0, i, 0)),
            pl.BlockSpec((MM_ROWS, 1), lambda i: (i, 0)),
            pl.BlockSpec((MM_ROWS, 1), lambda i: (i, 0)),
        ],
        out_shape=[
            jax.ShapeDtypeStruct((2, N, H), jnp.float32),
            jax.ShapeDtypeStruct((N, 1), jnp.float32),
            jax.ShapeDtypeStruct((N, 1), jnp.float32),
        ],
    )(an, ap, adp, den, b, W, a_s, a_d)


def _fn_body(an_ref, ap_ref, adp_ref, den_ref, b_ref, y_ref):
    adp = adp_ref[...]
    z = (jnp.exp(0.2 * adp) * an_ref[...] +
         jnp.exp(adp) * ap_ref[...]) / den_ref[...]
    z = jnp.maximum(z + b_ref[...], 0.0)
    n2 = jnp.sum(z * z, axis=1, keepdims=True)
    nrm = jnp.maximum(jnp.sqrt(n2), 1e-12)
    y_ref[...] = z / nrm


def _final_norm(an, ap, adp, den, b):
    H = an.shape[1]
    return pl.pallas_call(
        _fn_body,
        grid=(N // MM_ROWS,),
        in_specs=[
            pl.BlockSpec((MM_ROWS, H), lambda i: (i, 0)),
            pl.BlockSpec((MM_ROWS, H), lambda i: (i, 0)),
            pl.BlockSpec((MM_ROWS, 1), lambda i: (i, 0)),
            pl.BlockSpec((MM_ROWS, 1), lambda i: (i, 0)),
            pl.BlockSpec((1, H), lambda i: (0, 0)),
        ],
        out_specs=pl.BlockSpec((MM_ROWS, H), lambda i: (i, 0)),
        out_shape=jax.ShapeDtypeStruct((N, H), jnp.float32),
    )(an, ap, adp, den, b)


@functools.partial(
    pl.kernel,
    out_type=[jax.ShapeDtypeStruct((NW, NB, 128), jnp.int32),
              jax.ShapeDtypeStruct((NW, NB, 128), jnp.int32),
              jax.ShapeDtypeStruct((NW, N_PAD), jnp.float32)],
    mesh=_SC_MESH,
    compiler_params=_SC_PARAMS,
    scratch_types=[
        pltpu.VMEM((NB, 128), jnp.int32),
        pltpu.VMEM((NB, 128), jnp.int32),
        pltpu.VMEM((N,), jnp.float32),
        pltpu.VMEM((N,), jnp.float32),
        pltpu.VMEM((N_PAD,), jnp.float32),
    ],
)
def _sc_edge(src_hbm, dst_hbm, as_hbm, ad_hbm, srcp_hbm, dstp_hbm, part_hbm,
             src_v, dst_v, as_v, ad_v, den_v):
    wid = lax.axis_index("s") * 2 + lax.axis_index("c")
    base = wid * TPW
    pltpu.sync_copy(src_hbm.at[wid], src_v)
    pltpu.sync_copy(dst_hbm.at[wid], dst_v)
    pltpu.sync_copy(as_hbm, as_v)
    pltpu.sync_copy(ad_hbm, ad_v)

    zeros = jnp.zeros((16,), jnp.float32)

    def zero_body(i, c):
        den_v[pl.ds(i * 16, 16)] = zeros
        return c
    lax.fori_loop(0, N_PAD // 16, zero_body, 0)

    iota = lax.broadcasted_iota(jnp.int32, (16,), 0)
    nvec = jnp.full((16,), N, jnp.int32)
    zvec = jnp.zeros((16,), jnp.int32)

    def body(g, c):
        b = g // 8
        off = (g % 8) * 16
        s = src_v[b, pl.ds(off, 16)]
        d = dst_v[b, pl.ds(off, 16)]
        e = plsc.load_gather(as_v, [s]) + plsc.load_gather(ad_v, [d])
        pos = e > 0
        ex = jnp.exp(jnp.where(pos, e, 0.2 * e))
        gi = base + g * 16 + iota
        valid = gi < E_TOT
        ex = jnp.where(valid, ex, 0.0)
        bump = jnp.where(pos, nvec, zvec)
        src_v[b, pl.ds(off, 16)] = s + bump
        dst_v[b, pl.ds(off, 16)] = jnp.where(
            valid, d + bump, 2 * N + off + iota)
        plsc.addupdate_scatter(den_v, [d], ex)
        return c
    lax.fori_loop(0, TPW // 16, body, 0)

    pltpu.sync_copy(src_v, srcp_hbm.at[wid])
    pltpu.sync_copy(dst_v, dstp_hbm.at[wid])
    pltpu.sync_copy(den_v, part_hbm.at[wid])


@functools.partial(
    pl.kernel,
    out_type=jax.ShapeDtypeStruct((N_PAD,), jnp.float32),
    mesh=_SC_MESH,
    compiler_params=_SC_PARAMS,
    scratch_types=[
        pltpu.VMEM((NW, 320), jnp.float32),
        pltpu.VMEM((320,), jnp.float32),
    ],
)
def _sc_den_reduce(part_hbm, den_hbm, buf_v, out_v):
    wid = lax.axis_index("s") * 2 + lax.axis_index("c")
    col = wid * 320

    def cp(p, c):
        pltpu.sync_copy(part_hbm.at[p, pl.ds(col, 320)], buf_v.at[p])
        return c
    lax.fori_loop(0, NW, cp, 0)

    def cbody(cc, c):
        off = cc * 16

        def pbody(p, a):
            return a + buf_v[p, pl.ds(off, 16)]
        acc = lax.fori_loop(0, NW, pbody, jnp.zeros((16,), jnp.float32))
        out_v[pl.ds(off, 16)] = acc
        return c
    lax.fori_loop(0, 320 // 16, cbody, 0)
    pltpu.sync_copy(out_v, den_hbm.at[pl.ds(col, 320)])


_SPMM_CACHE = {}


def _make_spmm(n_chunk):
    if n_chunk in _SPMM_CACHE:
        return _SPMM_CACHE[n_chunk]
    cpc = n_chunk // 2

    @functools.partial(
        pl.kernel,
        out_type=jax.ShapeDtypeStruct((R_ROWS, n_chunk, CW), jnp.float32),
        mesh=_SC_MESH,
        compiler_params=_SC_PARAMS,
        scratch_types=[
            pltpu.VMEM((2 * NB, 128), jnp.int32),
            pltpu.VMEM((2 * NB, 128), jnp.int32),
            pltpu.VMEM((2, 128, 1, CW), jnp.float32),
            pltpu.VMEM((63, 1, CW), jnp.float32),
            pltpu.VMEM_SHARED((R_ROWS, 1, CW), jnp.float32),
            pltpu.SemaphoreType.DMA,
            pltpu.SemaphoreType.DMA,
        ],
    )
    def spmm(srcp_hbm, dstp_hbm, u_hbm, out_hbm,
             idx_v, dstp_v, gbuf, zbuf, acc, sem0, sem1):
        cid = lax.axis_index("c")
        sid = lax.axis_index("s")

        zeros = jnp.zeros((16,), jnp.float32)

        def zb(i, c):
            zbuf[i // 4, 0, pl.ds((i % 4) * 16, 16)] = zeros
            return c
        lax.fori_loop(0, 63 * 4, zb, 0)

        pltpu.sync_copy(srcp_hbm.at[sid * 2], idx_v.at[pl.ds(0, NB)])
        pltpu.sync_copy(srcp_hbm.at[sid * 2 + 1], idx_v.at[pl.ds(NB, NB)])
        pltpu.sync_copy(dstp_hbm.at[sid * 2], dstp_v.at[pl.ds(0, NB)])
        pltpu.sync_copy(dstp_hbm.at[sid * 2 + 1], dstp_v.at[pl.ds(NB, NB)])

        c0 = cid * cpc - 1

        def ib(i, c):
            b = i // 8
            off = (i % 8) * 16
            idx_v[b, pl.ds(off, 16)] = idx_v[b, pl.ds(off, 16)] * n_chunk + c0
            return c
        lax.fori_loop(0, 2 * NB * 8, ib, 0)

        row0 = sid * RPT

        def chunk_body(t, c):
            k = cid * cpc + t

            def zr(r, c2):
                pltpu.sync_copy(zbuf, acc.at[pl.ds(row0 + r * 63, 63)])
                return c2
            lax.fori_loop(0, RPT // 63, zr, 0)

            def bump(i, c2):
                b = i // 8
                off = (i % 8) * 16
                idx_v[b, pl.ds(off, 16)] = idx_v[b, pl.ds(off, 16)] + 1
                return c2
            lax.fori_loop(0, 2 * NB * 8, bump, 0)

            plsc.subcore_barrier()

            def pair(j, c2):
                b0 = j * 2
                b1 = b0 + 1
                cp0 = pltpu.async_copy(u_hbm.at[idx_v.at[b0]],
                                       gbuf.at[0], sem0)
                cp1 = pltpu.async_copy(u_hbm.at[idx_v.at[b1]],
                                       gbuf.at[1], sem1)
                cp0.wait()
                pltpu.sync_copy(gbuf.at[0], acc.at[dstp_v.at[b0]], add=True)
                cp1.wait()
                pltpu.sync_copy(gbuf.at[1], acc.at[dstp_v.at[b1]], add=True)
                return c2
            lax.fori_loop(0, NB, pair, 0)

            plsc.subcore_barrier()

            def wr(r, c2):
                rr = row0 + r * (RPT // 2)
                pltpu.sync_copy(acc.at[pl.ds(rr, RPT // 2)],
                                out_hbm.at[pl.ds(rr, RPT // 2), pl.ds(k, 1)])
                return c2
            lax.fori_loop(0, 2, wr, 0)

            plsc.subcore_barrier()
            return c
        lax.fori_loop(0, cpc, chunk_body, 0)

    _SPMM_CACHE[n_chunk] = spmm
    return spmm


def _edge_phase(src3, dst3, aso, ado, H):
    n_chunk = H // CW
    srcp, dstp, parts = _sc_edge(src3, dst3, aso.reshape(N), ado.reshape(N))
    den = _sc_den_reduce(parts)
    return srcp, dstp, den[:N].reshape(N, 1), n_chunk


def _spmm_phase(srcp, dstp, u, n_chunk):
    H = n_chunk * CW
    out = _make_spmm(n_chunk)(srcp, dstp,
                              u.reshape(2 * N * n_chunk, 1, CW))
    out_r = out.reshape(R_ROWS, H)
    return out_r[:N], out_r[N:2 * N]


def kernel(x, _, edge_index, W1, a_src1, a_dst1, b1, W2, a_src2, a_dst2, b2,
           W3, a_src3, a_dst3, b3):
    loop = jnp.arange(N, dtype=jnp.int32)
    padz = jnp.zeros((E_PAD - E_TOT,), jnp.int32)
    src3 = jnp.concatenate([edge_index[0], loop, padz]).reshape(NW, NB, 128)
    dst3 = jnp.concatenate([edge_index[1], loop, padz]).reshape(NW, NB, 128)

    H1, H2, H3 = W1.shape[1], W2.shape[1], W3.shape[1]

    u, aso, ado1 = _matmul_first(x, W1, a_src1.reshape(H1, 1),
                                 a_dst1.reshape(H1, 1))
    srcp, dstp, den1, nc = _edge_phase(src3, dst3, aso, ado1, H1)
    an1, ap1 = _spmm_phase(srcp, dstp, u, nc)

    u, aso, ado2 = _matmul_mid(an1, ap1, ado1, den1, b1.reshape(1, -1), W2,
                               a_src2.reshape(H2, 1), a_dst2.reshape(H2, 1))
    srcp, dstp, den2, nc = _edge_phase(src3, dst3, aso, ado2, H2)
    an2, ap2 = _spmm_phase(srcp, dstp, u, nc)

    u, aso, ado3 = _matmul_mid(an2, ap2, ado2, den2, b2.reshape(1, -1), W3,
                               a_src3.reshape(H3, 1), a_dst3.reshape(H3, 1))
    srcp, dstp, den3, nc = _edge_phase(src3, dst3, aso, ado3, H3)
    an3, ap3 = _spmm_phase(srcp, dstp, u, nc)

    return _final_norm(an3, ap3, ado3, den3, b3.reshape(1, -1))
